# baseline (speedup 1.0000x reference)
import functools

import jax
import jax.numpy as jnp
from jax import lax
from jax.experimental import pallas as pl
from jax.experimental.pallas import tpu as pltpu

_NBUF = 6
_ROWS = 80


def _gc_kernel(v_ref, w_ref, b_ref, ad_hbm, aa_hbm, o_hbm,
               s_ref, ad_buf, aa_buf, o_stage,
               ad_sem, aa_sem, o_sem, *, n_chunks):
    def _start(chunk, slot):
        pltpu.make_async_copy(
            ad_hbm.at[pl.ds(chunk * _ROWS, _ROWS), :],
            ad_buf.at[slot], ad_sem.at[slot]).start()
        pltpu.make_async_copy(
            aa_hbm.at[pl.ds(chunk * _ROWS, _ROWS), :],
            aa_buf.at[slot], aa_sem.at[slot]).start()

    def _out_copy(chunk, slot):
        return pltpu.make_async_copy(
            o_stage.at[slot],
            o_hbm.at[pl.ds(chunk * _ROWS, _ROWS), :], o_sem.at[slot])

    for slot in range(_NBUF):
        _start(slot, slot)

    s_ref[...] = jnp.dot(v_ref[...].astype(jnp.bfloat16),
                         w_ref[...].astype(jnp.bfloat16),
                         preferred_element_type=jnp.float32
                         ).astype(jnp.bfloat16)

    def _body(i, carry):
        slot = lax.rem(i, _NBUF)
        pltpu.make_async_copy(
            ad_hbm.at[pl.ds(i * _ROWS, _ROWS), :],
            ad_buf.at[slot], ad_sem.at[slot]).wait()
        pltpu.make_async_copy(
            aa_hbm.at[pl.ds(i * _ROWS, _ROWS), :],
            aa_buf.at[slot], aa_sem.at[slot]).wait()

        @pl.when(i >= _NBUF)
        def _():
            _out_copy(i - _NBUF, slot).wait()

        a = (ad_buf[slot] + aa_buf[slot]).astype(jnp.bfloat16)
        o_stage[slot] = (
            jnp.dot(a, s_ref[...], preferred_element_type=jnp.float32)
            + b_ref[...])
        _out_copy(i, slot).start()

        @pl.when(i + _NBUF < n_chunks)
        def _():
            _start(i + _NBUF, slot)

        return carry

    lax.fori_loop(0, n_chunks, _body, 0, unroll=False)

    def _drain(j, carry):
        _out_copy(j, lax.rem(j, _NBUF)).wait()
        return carry

    lax.fori_loop(n_chunks - _NBUF, n_chunks, _drain, 0, unroll=False)


def kernel(vertex, adj_distance, adj_angle, weights, bias):
    n, in_f = vertex.shape
    out_f = weights.shape[1]
    bias2 = bias.reshape(1, out_f)
    n_chunks = n // _ROWS

    return pl.pallas_call(
        functools.partial(_gc_kernel, n_chunks=n_chunks),
        in_specs=[
            pl.BlockSpec(memory_space=pltpu.VMEM),
            pl.BlockSpec(memory_space=pltpu.VMEM),
            pl.BlockSpec(memory_space=pltpu.VMEM),
            pl.BlockSpec(memory_space=pl.ANY),
            pl.BlockSpec(memory_space=pl.ANY),
        ],
        out_specs=pl.BlockSpec(memory_space=pl.ANY),
        out_shape=jax.ShapeDtypeStruct((n, out_f), jnp.float32),
        scratch_shapes=[
            pltpu.VMEM((n, out_f), jnp.bfloat16),
            pltpu.VMEM((_NBUF, _ROWS, n), jnp.float32),
            pltpu.VMEM((_NBUF, _ROWS, n), jnp.float32),
            pltpu.VMEM((_NBUF, _ROWS, out_f), jnp.float32),
            pltpu.SemaphoreType.DMA((_NBUF,)),
            pltpu.SemaphoreType.DMA((_NBUF,)),
            pltpu.SemaphoreType.DMA((_NBUF,)),
        ],
    )(vertex, weights, bias2, adj_distance, adj_angle)


# speedup vs baseline: 1.0278x; 1.0278x over previous
import functools

import jax
import jax.numpy as jnp
from jax import lax
from jax.experimental import pallas as pl
from jax.experimental.pallas import tpu as pltpu

_NBUF = 4
_ROWS = 80


def _gc_kernel(v_ref, w_ref, b_ref, ad_hbm, aa_hbm, o_hbm,
               s_ref, ad_buf, aa_buf, o_stage,
               ad_sem, aa_sem, o_sem, *, n_chunks):
    def _start(chunk, slot):
        pltpu.make_async_copy(
            ad_hbm.at[pl.ds(chunk * _ROWS, _ROWS), :],
            ad_buf.at[slot], ad_sem.at[slot]).start()
        pltpu.make_async_copy(
            aa_hbm.at[pl.ds(chunk * _ROWS, _ROWS), :],
            aa_buf.at[slot], aa_sem.at[slot]).start()

    def _out_copy(chunk, slot):
        return pltpu.make_async_copy(
            o_stage.at[slot],
            o_hbm.at[pl.ds(chunk * _ROWS, _ROWS), :], o_sem.at[slot])

    for slot in range(_NBUF):
        _start(slot, slot)

    s_ref[...] = jnp.dot(v_ref[...].astype(jnp.bfloat16),
                         w_ref[...].astype(jnp.bfloat16),
                         preferred_element_type=jnp.float32
                         ).astype(jnp.bfloat16)

    def _body(i, carry):
        slot = lax.rem(i, _NBUF)
        pltpu.make_async_copy(
            ad_hbm.at[pl.ds(i * _ROWS, _ROWS), :],
            ad_buf.at[slot], ad_sem.at[slot]).wait()
        pltpu.make_async_copy(
            aa_hbm.at[pl.ds(i * _ROWS, _ROWS), :],
            aa_buf.at[slot], aa_sem.at[slot]).wait()

        @pl.when(i >= _NBUF)
        def _():
            _out_copy(i - _NBUF, slot).wait()

        a = (ad_buf[slot] + aa_buf[slot]).astype(jnp.bfloat16)
        o_stage[slot] = (
            jnp.dot(a, s_ref[...], preferred_element_type=jnp.float32)
            + b_ref[...])
        _out_copy(i, slot).start()

        @pl.when(i + _NBUF < n_chunks)
        def _():
            _start(i + _NBUF, slot)

        return carry

    lax.fori_loop(0, n_chunks, _body, 0, unroll=False)

    def _drain(j, carry):
        _out_copy(j, lax.rem(j, _NBUF)).wait()
        return carry

    lax.fori_loop(n_chunks - _NBUF, n_chunks, _drain, 0, unroll=False)


def kernel(vertex, adj_distance, adj_angle, weights, bias):
    n, in_f = vertex.shape
    out_f = weights.shape[1]
    bias2 = bias.reshape(1, out_f)
    n_chunks = n // _ROWS

    return pl.pallas_call(
        functools.partial(_gc_kernel, n_chunks=n_chunks),
        in_specs=[
            pl.BlockSpec(memory_space=pltpu.VMEM),
            pl.BlockSpec(memory_space=pltpu.VMEM),
            pl.BlockSpec(memory_space=pltpu.VMEM),
            pl.BlockSpec(memory_space=pl.ANY),
            pl.BlockSpec(memory_space=pl.ANY),
        ],
        out_specs=pl.BlockSpec(memory_space=pl.ANY),
        out_shape=jax.ShapeDtypeStruct((n, out_f), jnp.float32),
        scratch_shapes=[
            pltpu.VMEM((n, out_f), jnp.bfloat16),
            pltpu.VMEM((_NBUF, _ROWS, n), jnp.float32),
            pltpu.VMEM((_NBUF, _ROWS, n), jnp.float32),
            pltpu.VMEM((_NBUF, _ROWS, out_f), jnp.float32),
            pltpu.SemaphoreType.DMA((_NBUF,)),
            pltpu.SemaphoreType.DMA((_NBUF,)),
            pltpu.SemaphoreType.DMA((_NBUF,)),
        ],
    )(vertex, weights, bias2, adj_distance, adj_angle)
